# Initial kernel scaffold; baseline (speedup 1.0000x reference)
import jax
import jax.numpy as jnp
from jax.experimental import pallas as pl


def kernel(word_input, weight_all):
    raise NotImplementedError("write your pallas kernel here")


# speedup vs baseline: 4.0921x; 4.0921x over previous
import functools

import jax
import jax.numpy as jnp
from jax import lax
from jax.experimental import pallas as pl
from jax.experimental.pallas import tpu as pltpu
from jax.experimental.pallas import tpu_sc as plsc

BATCH = 4096
SEQ = 200
EMBED = 64
N = BATCH * SEQ
NUM_WORKERS = 32
PER_W = N // NUM_WORKERS
CHUNK = 512
NCHUNK = PER_W // CHUNK

_mesh = plsc.VectorSubcoreMesh(core_axis_name="c", subcore_axis_name="s")


@functools.partial(
    pl.kernel,
    out_type=jax.ShapeDtypeStruct((N, EMBED), jnp.float32),
    mesh=_mesh,
    compiler_params=pltpu.CompilerParams(use_tc_tiling_on_sc=False),
    scratch_types=[
        pltpu.VMEM((PER_W,), jnp.int32),
        pltpu.VMEM((CHUNK, EMBED), jnp.float32),
        pltpu.SemaphoreType.DMA,
    ],
)
def _embed_sc(idx_hbm, table_hbm, out_hbm, idx_v, rows_v, sem):
    wid = lax.axis_index("s") * 2 + lax.axis_index("c")
    base = wid * PER_W
    pltpu.sync_copy(idx_hbm.at[pl.ds(base, PER_W)], idx_v)

    def body(i, carry):
        off = pl.multiple_of(i * CHUNK, 8)
        pltpu.async_copy(table_hbm.at[idx_v.at[pl.ds(off, CHUNK)]], rows_v, sem).wait()
        pltpu.sync_copy(rows_v, out_hbm.at[pl.ds(base + off, CHUNK)])
        return carry

    lax.fori_loop(0, NCHUNK, body, 0)


def kernel(word_input, weight_all):
    idx = word_input.reshape(N).astype(jnp.int32)
    out = _embed_sc(idx, weight_all)
    return out.reshape(BATCH, SEQ, EMBED)


# speedup vs baseline: 4.2473x; 1.0379x over previous
import functools

import jax
import jax.numpy as jnp
from jax import lax
from jax.experimental import pallas as pl
from jax.experimental.pallas import tpu as pltpu
from jax.experimental.pallas import tpu_sc as plsc

BATCH = 4096
SEQ = 200
EMBED = 64
N = BATCH * SEQ
NUM_WORKERS = 32
PER_W = N // NUM_WORKERS
CHUNK = 512
NCHUNK = PER_W // CHUNK
NH = NCHUNK // 2

_mesh = plsc.VectorSubcoreMesh(core_axis_name="c", subcore_axis_name="s")


@functools.partial(
    pl.kernel,
    out_type=jax.ShapeDtypeStruct((N, EMBED), jnp.float32),
    mesh=_mesh,
    compiler_params=pltpu.CompilerParams(use_tc_tiling_on_sc=False),
    scratch_types=[
        pltpu.VMEM((PER_W,), jnp.int32),
        pltpu.VMEM((CHUNK, EMBED), jnp.float32),
        pltpu.VMEM((CHUNK, EMBED), jnp.float32),
        pltpu.SemaphoreType.DMA,
        pltpu.SemaphoreType.DMA,
        pltpu.SemaphoreType.DMA,
        pltpu.SemaphoreType.DMA,
    ],
)
def _embed_sc(idx_hbm, table_hbm, out_hbm, idx_v, buf_a, buf_b,
              gsem_a, gsem_b, ssem_a, ssem_b):
    wid = lax.axis_index("s") * 2 + lax.axis_index("c")
    base = wid * PER_W
    pltpu.sync_copy(idx_hbm.at[pl.ds(base, PER_W)], idx_v)

    def gather(chunk, buf, sem):
        off = pl.multiple_of(chunk * CHUNK, 8)
        return pltpu.async_copy(table_hbm.at[idx_v.at[pl.ds(off, CHUNK)]], buf, sem)

    def store(chunk, buf, sem):
        off = pl.multiple_of(base + chunk * CHUNK, 8)
        return pltpu.async_copy(buf, out_hbm.at[pl.ds(off, CHUNK)], sem)

    def wait_gather(buf, sem):
        pltpu.make_async_copy(table_hbm.at[pl.ds(0, CHUNK)], buf, sem).wait()

    def wait_store(buf, sem):
        pltpu.make_async_copy(buf, out_hbm.at[pl.ds(base, CHUNK)], sem).wait()

    gather(0, buf_a, gsem_a)

    def body(i, carry):
        g = 2 * i
        wait_gather(buf_a, gsem_a)

        @pl.when(i > 0)
        def _():
            wait_store(buf_b, ssem_b)

        gather(g + 1, buf_b, gsem_b)
        store(g, buf_a, ssem_a)
        wait_gather(buf_b, gsem_b)
        wait_store(buf_a, ssem_a)

        @pl.when(i + 1 < NH)
        def _():
            gather(g + 2, buf_a, gsem_a)

        store(g + 1, buf_b, ssem_b)
        return carry

    lax.fori_loop(0, NH, body, 0)
    wait_store(buf_b, ssem_b)


def kernel(word_input, weight_all):
    idx = word_input.reshape(N).astype(jnp.int32)
    out = _embed_sc(idx, weight_all)
    return out.reshape(BATCH, SEQ, EMBED)


# speedup vs baseline: 4.2587x; 1.0027x over previous
import functools

import jax
import jax.numpy as jnp
from jax import lax
from jax.experimental import pallas as pl
from jax.experimental.pallas import tpu as pltpu
from jax.experimental.pallas import tpu_sc as plsc

BATCH = 4096
SEQ = 200
EMBED = 64
N = BATCH * SEQ
NUM_WORKERS = 32
PER_W = N // NUM_WORKERS
NSLOT = 8
CHUNK = 128
NROUND = PER_W // (NSLOT * CHUNK)

_mesh = plsc.VectorSubcoreMesh(core_axis_name="c", subcore_axis_name="s")


@functools.partial(
    pl.kernel,
    out_type=jax.ShapeDtypeStruct((N, EMBED), jnp.float32),
    mesh=_mesh,
    compiler_params=pltpu.CompilerParams(use_tc_tiling_on_sc=False),
    scratch_types=[
        pltpu.VMEM((PER_W,), jnp.int32),
        pltpu.VMEM((NSLOT, CHUNK, EMBED), jnp.float32),
        pltpu.SemaphoreType.DMA((NSLOT,)),
        pltpu.SemaphoreType.DMA((NSLOT,)),
    ],
)
def _embed_sc(idx_hbm, table_hbm, out_hbm, idx_v, bufs, gsems, ssems):
    wid = lax.axis_index("s") * 2 + lax.axis_index("c")
    base = wid * PER_W
    pltpu.sync_copy(idx_hbm.at[pl.ds(base, PER_W)], idx_v)

    def body(r, carry):
        for s in range(NSLOT):
            chunk_off = pl.multiple_of((r * NSLOT + s) * CHUNK, 8)

            @pl.when(r > 0)
            def _():
                pltpu.make_async_copy(
                    bufs.at[s], out_hbm.at[pl.ds(base, CHUNK)], ssems.at[s]
                ).wait()

            pltpu.async_copy(
                table_hbm.at[idx_v.at[pl.ds(chunk_off, CHUNK)]],
                bufs.at[s], gsems.at[s])
        for s in range(NSLOT):
            chunk_off = pl.multiple_of((r * NSLOT + s) * CHUNK, 8)
            pltpu.make_async_copy(
                table_hbm.at[pl.ds(0, CHUNK)], bufs.at[s], gsems.at[s]
            ).wait()
            pltpu.async_copy(
                bufs.at[s], out_hbm.at[pl.ds(base + chunk_off, CHUNK)],
                ssems.at[s])
        return carry

    lax.fori_loop(0, NROUND, body, 0)
    for s in range(NSLOT):
        pltpu.make_async_copy(
            bufs.at[s], out_hbm.at[pl.ds(base, CHUNK)], ssems.at[s]
        ).wait()


def kernel(word_input, weight_all):
    idx = word_input.reshape(N).astype(jnp.int32)
    out = _embed_sc(idx, weight_all)
    return out.reshape(BATCH, SEQ, EMBED)
